# Initial kernel scaffold; baseline (speedup 1.0000x reference)
import functools

import jax
import jax.numpy as jnp
from jax import lax
from jax.experimental import pallas as pl
from jax.experimental.pallas import tpu as pltpu
from jax.experimental.pallas import tpu_sc as plsc

BS = 8
V = 2048
F = 256
K = 16
POOL = 512

NC = 2
NS = 16
NW = NC * NS
NQ = BS * POOL
QPW = NQ // NW
CHUNK_Q = 8
N_CHUNK = QPW // CHUNK_Q
ROWS_PER_CHUNK = CHUNK_Q * K


def _knn_body(qv_ref, vt_ref, out_ref):
    qv = qv_ref[0]
    vt = vt_ref[0]
    qx, qy, qz = qv[:, 0:1], qv[:, 1:2], qv[:, 2:3]
    vx, vy, vz = vt[0:1, :], vt[1:2, :], vt[2:3, :]
    inner = qx * vx + qy * vy + qz * vz
    quad_v = vx * vx + vy * vy + vz * vz
    quad_q = qx * qx + qy * qy + qz * qz
    d = inner * -2.0 + quad_v + quad_q

    iota = lax.broadcasted_iota(jnp.int32, (POOL, V), 1)
    base = pl.program_id(0) * V
    cols = []
    for t in range(K + 1):
        m = jnp.min(d, axis=1, keepdims=True)
        amin = jnp.min(jnp.where(d == m, iota, V), axis=1, keepdims=True)
        if t > 0:
            cols.append(amin + base)
        if t < K:
            d = jnp.where(iota == amin, jnp.float32(jnp.inf), d)
    out_ref[0] = jnp.concatenate(cols, axis=1)


def _knn_indices(qv, vt):
    return pl.pallas_call(
        _knn_body,
        grid=(BS,),
        in_specs=[
            pl.BlockSpec((1, POOL, 3), lambda b: (b, 0, 0)),
            pl.BlockSpec((1, 3, V), lambda b: (b, 0, 0)),
        ],
        out_specs=pl.BlockSpec((1, POOL, K), lambda b: (b, 0, 0)),
        out_shape=jax.ShapeDtypeStruct((BS, POOL, K), jnp.int32),
    )(qv, vt)


@functools.partial(
    pl.kernel,
    out_type=jax.ShapeDtypeStruct((NQ, F), jnp.float32),
    mesh=plsc.VectorSubcoreMesh(core_axis_name="c", subcore_axis_name="s"),
    scratch_types=[
        pltpu.VMEM((QPW * K,), jnp.int32),
        pltpu.VMEM((ROWS_PER_CHUNK, F), jnp.float32),
        pltpu.VMEM((QPW, F), jnp.float32),
        pltpu.SemaphoreType.DMA,
    ],
)
def _sc_pool(fm_hbm, idx_hbm, out_hbm, idx_v, rows_v, out_v, sem):
    wid = lax.axis_index("s") * NC + lax.axis_index("c")
    qbase = pl.multiple_of(wid * QPW, QPW)
    pltpu.sync_copy(idx_hbm.at[pl.ds(qbase * K, QPW * K)], idx_v)

    def chunk_body(ci, carry):
        off = pl.multiple_of(ci * ROWS_PER_CHUNK, ROWS_PER_CHUNK)
        pltpu.async_copy(
            fm_hbm.at[idx_v.at[pl.ds(off, ROWS_PER_CHUNK)]], rows_v, sem
        ).wait()

        def q_body(qq, inner_carry):
            row0 = qq * K
            for c in range(F // 16):
                sl = pl.ds(c * 16, 16)
                acc = rows_v[row0, sl]
                for r in range(1, K):
                    acc = jnp.maximum(acc, rows_v[row0 + r, sl])
                out_v[ci * CHUNK_Q + qq, sl] = acc
            return inner_carry

        lax.fori_loop(0, CHUNK_Q, q_body, 0)
        return carry

    lax.fori_loop(0, N_CHUNK, chunk_body, 0)
    pltpu.sync_copy(out_v, out_hbm.at[pl.ds(qbase, QPW)])


def kernel(vertices, feature_map):
    sample_idx = jax.random.permutation(jax.random.key(42), V)[:POOL]
    qv = jnp.take(vertices, sample_idx, axis=1)
    vt = vertices.transpose(0, 2, 1)
    gidx = _knn_indices(qv, vt)
    fm_flat = feature_map.reshape(BS * V, F)
    idx_flat = gidx.reshape(NQ * K)
    pooled = _sc_pool(fm_flat, idx_flat)
    return (qv, pooled.reshape(BS, POOL, F))


# speedup vs baseline: 41.9948x; 41.9948x over previous
import functools

import jax
import jax.numpy as jnp
from jax import lax
from jax.experimental import pallas as pl
from jax.experimental.pallas import tpu as pltpu
from jax.experimental.pallas import tpu_sc as plsc

BS = 8
V = 2048
F = 256
K = 16
POOL = 512

NC = 2
NS = 16
NW = NC * NS
NQ = BS * POOL
QPW = NQ // NW
CHUNK_Q = 8
N_CHUNK = QPW // CHUNK_Q
ROWS_PER_CHUNK = CHUNK_Q * K


def _knn_body(qv_ref, vt_ref, out_ref):
    qv = qv_ref[0]
    vt = vt_ref[0]
    inner = jax.lax.dot_general(
        qv.astype(jnp.bfloat16), vt.astype(jnp.bfloat16),
        (((1,), (0,)), ((), ())),
        preferred_element_type=jnp.float32,
    )
    qx, qy, qz = qv[:, 0:1], qv[:, 1:2], qv[:, 2:3]
    vx, vy, vz = vt[0:1, :], vt[1:2, :], vt[2:3, :]
    quad_v = vx * vx + vy * vy + vz * vz
    quad_q = qx * qx + qy * qy + qz * qz
    d = inner * -2.0 + quad_v + quad_q

    iota = lax.broadcasted_iota(jnp.int32, (POOL, V), 1)
    base = pl.program_id(0) * V
    cols = []
    for t in range(K + 1):
        m = jnp.min(d, axis=1, keepdims=True)
        amin = jnp.min(jnp.where(d == m, iota, V), axis=1, keepdims=True)
        if t > 0:
            cols.append(amin + base)
        if t < K:
            d = jnp.where(iota == amin, jnp.float32(jnp.inf), d)
    out_ref[0] = jnp.concatenate(cols, axis=1)


def _knn_indices(qv, vt):
    return pl.pallas_call(
        _knn_body,
        grid=(BS,),
        in_specs=[
            pl.BlockSpec((1, POOL, 3), lambda b: (b, 0, 0)),
            pl.BlockSpec((1, 3, V), lambda b: (b, 0, 0)),
        ],
        out_specs=pl.BlockSpec((1, POOL, K), lambda b: (b, 0, 0)),
        out_shape=jax.ShapeDtypeStruct((BS, POOL, K), jnp.int32),
    )(qv, vt)


def _sc_pool_body(fm_hbm, idx_hbm, out_hbm, idx_v, rows_v, out_v, sem):
    wid = lax.axis_index("s") * NC + lax.axis_index("c")
    qbase = pl.multiple_of(wid * QPW, QPW)
    pltpu.sync_copy(idx_hbm.at[pl.ds(qbase * K, QPW * K)], idx_v)

    def chunk_body(ci, carry):
        off = pl.multiple_of(ci * ROWS_PER_CHUNK, ROWS_PER_CHUNK)
        pltpu.async_copy(
            fm_hbm.at[idx_v.at[pl.ds(off, ROWS_PER_CHUNK)]], rows_v, sem
        ).wait()

        def q_body(qq, inner_carry):
            row0 = qq * K
            for c in range(F // 16):
                sl = pl.ds(c * 16, 16)
                acc = rows_v[row0, sl]
                for r in range(1, K):
                    acc = jnp.maximum(acc, rows_v[row0 + r, sl])
                out_v[ci * CHUNK_Q + qq, sl] = acc
            return inner_carry

        lax.fori_loop(0, CHUNK_Q, q_body, 0)
        return carry

    lax.fori_loop(0, N_CHUNK, chunk_body, 0)
    pltpu.sync_copy(out_v, out_hbm.at[pl.ds(qbase, QPW)])


@functools.cache
def _make_sc_pool():
    return functools.partial(
        pl.kernel,
        out_type=jax.ShapeDtypeStruct((NQ, F), jnp.float32),
        mesh=plsc.VectorSubcoreMesh(core_axis_name="c", subcore_axis_name="s"),
        scratch_types=[
            pltpu.VMEM((QPW * K,), jnp.int32),
            pltpu.VMEM((ROWS_PER_CHUNK, F), jnp.float32),
            pltpu.VMEM((QPW, F), jnp.float32),
            pltpu.SemaphoreType.DMA,
        ],
    )(_sc_pool_body)


def kernel(vertices, feature_map):
    sample_idx = jax.random.permutation(jax.random.key(42), V)[:POOL]
    qv = jnp.take(vertices, sample_idx, axis=1)
    vt = vertices.transpose(0, 2, 1)
    gidx = _knn_indices(qv, vt)
    fm_flat = feature_map.reshape(BS * V, F)
    idx_flat = gidx.reshape(NQ * K)
    pooled = _make_sc_pool()(fm_flat, idx_flat)
    return (qv, pooled.reshape(BS, POOL, F))


# speedup vs baseline: 46.5677x; 1.1089x over previous
import functools

import jax
import jax.numpy as jnp
from jax import lax
from jax.experimental import pallas as pl
from jax.experimental.pallas import tpu as pltpu
from jax.experimental.pallas import tpu_sc as plsc

BS = 8
V = 2048
F = 256
K = 16
POOL = 512

NC = 2
NS = 16
NW = NC * NS
NQ = BS * POOL
QPW = NQ // NW
CHUNK_Q = 8
N_CHUNK = QPW // CHUNK_Q
ROWS_PER_CHUNK = CHUNK_Q * K


def _knn_body(qv_ref, vt_ref, out_ref):
    qv = qv_ref[0]
    vt = vt_ref[0]
    inner = jax.lax.dot_general(
        qv.astype(jnp.bfloat16), vt.astype(jnp.bfloat16),
        (((1,), (0,)), ((), ())),
        preferred_element_type=jnp.float32,
    )
    qx, qy, qz = qv[:, 0:1], qv[:, 1:2], qv[:, 2:3]
    vx, vy, vz = vt[0:1, :], vt[1:2, :], vt[2:3, :]
    quad_v = vx * vx + vy * vy + vz * vz
    quad_q = qx * qx + qy * qy + qz * qz
    d = inner * -2.0 + quad_v + quad_q

    iota = lax.broadcasted_iota(jnp.int32, (POOL, V), 1).astype(jnp.float32)
    base = pl.program_id(0) * V
    cols = []
    for t in range(K + 1):
        m = jnp.min(d, axis=1, keepdims=True)
        amin = jnp.min(
            jnp.where(d == m, iota, jnp.float32(V)), axis=1, keepdims=True
        )
        if t > 0:
            cols.append(amin.astype(jnp.int32) + base)
        if t < K:
            d = jnp.where(iota == amin, jnp.float32(jnp.inf), d)
    out_ref[0] = jnp.concatenate(cols, axis=1)


def _knn_indices(qv, vt):
    return pl.pallas_call(
        _knn_body,
        grid=(BS,),
        in_specs=[
            pl.BlockSpec((1, POOL, 3), lambda b: (b, 0, 0)),
            pl.BlockSpec((1, 3, V), lambda b: (b, 0, 0)),
        ],
        out_specs=pl.BlockSpec((1, POOL, K), lambda b: (b, 0, 0)),
        out_shape=jax.ShapeDtypeStruct((BS, POOL, K), jnp.int32),
    )(qv, vt)


def _sc_pool_body(fm_hbm, idx_hbm, out_hbm, idx_v, rows_v, out_v, sem):
    wid = lax.axis_index("s") * NC + lax.axis_index("c")
    qbase = pl.multiple_of(wid * QPW, QPW)
    pltpu.sync_copy(idx_hbm.at[pl.ds(qbase * K, QPW * K)], idx_v)

    def chunk_body(ci, carry):
        off = pl.multiple_of(ci * ROWS_PER_CHUNK, ROWS_PER_CHUNK)
        pltpu.async_copy(
            fm_hbm.at[idx_v.at[pl.ds(off, ROWS_PER_CHUNK)]], rows_v, sem
        ).wait()

        def q_body(qq, inner_carry):
            row0 = qq * K
            for c in range(F // 16):
                sl = pl.ds(c * 16, 16)
                acc = rows_v[row0, sl]
                for r in range(1, K):
                    acc = jnp.maximum(acc, rows_v[row0 + r, sl])
                out_v[ci * CHUNK_Q + qq, sl] = acc
            return inner_carry

        lax.fori_loop(0, CHUNK_Q, q_body, 0)
        return carry

    lax.fori_loop(0, N_CHUNK, chunk_body, 0)
    pltpu.sync_copy(out_v, out_hbm.at[pl.ds(qbase, QPW)])


@functools.cache
def _make_sc_pool():
    return functools.partial(
        pl.kernel,
        out_type=jax.ShapeDtypeStruct((NQ, F), jnp.float32),
        mesh=plsc.VectorSubcoreMesh(core_axis_name="c", subcore_axis_name="s"),
        scratch_types=[
            pltpu.VMEM((QPW * K,), jnp.int32),
            pltpu.VMEM((ROWS_PER_CHUNK, F), jnp.float32),
            pltpu.VMEM((QPW, F), jnp.float32),
            pltpu.SemaphoreType.DMA,
        ],
    )(_sc_pool_body)


def kernel(vertices, feature_map):
    sample_idx = jax.random.permutation(jax.random.key(42), V)[:POOL]
    qv = jnp.take(vertices, sample_idx, axis=1)
    vt = vertices.transpose(0, 2, 1)
    gidx = _knn_indices(qv, vt)
    fm_flat = feature_map.reshape(BS * V, F)
    idx_flat = gidx.reshape(NQ * K)
    pooled = _make_sc_pool()(fm_flat, idx_flat)
    return (qv, pooled.reshape(BS, POOL, F))


# speedup vs baseline: 52.1185x; 1.1192x over previous
import functools

import jax
import jax.numpy as jnp
from jax import lax
from jax.experimental import pallas as pl
from jax.experimental.pallas import tpu as pltpu
from jax.experimental.pallas import tpu_sc as plsc

BS = 8
V = 2048
F = 256
K = 16
POOL = 512

NC = 2
NS = 16
NW = NC * NS
NQ = BS * POOL
QPW = NQ // NW
CHUNK_Q = 8
N_CHUNK = QPW // CHUNK_Q
ROWS_PER_CHUNK = CHUNK_Q * K


def _knn_body(qv_ref, vt_ref, out_ref):
    qv = qv_ref[0]
    vt = vt_ref[0]
    inner = jax.lax.dot_general(
        qv.astype(jnp.bfloat16), vt.astype(jnp.bfloat16),
        (((1,), (0,)), ((), ())),
        preferred_element_type=jnp.float32,
    )
    qx, qy, qz = qv[:, 0:1], qv[:, 1:2], qv[:, 2:3]
    vx, vy, vz = vt[0:1, :], vt[1:2, :], vt[2:3, :]
    quad_v = vx * vx + vy * vy + vz * vz
    quad_q = qx * qx + qy * qy + qz * qz
    d = inner * -2.0 + quad_v + quad_q

    iota = lax.broadcasted_iota(jnp.int32, (POOL, V), 1).astype(jnp.float32)
    base = pl.program_id(0) * V
    cols = []
    for t in range(K + 1):
        m = jnp.min(d, axis=1, keepdims=True)
        amin = jnp.min(
            jnp.where(d == m, iota, jnp.float32(V)), axis=1, keepdims=True
        )
        if t > 0:
            cols.append(amin.astype(jnp.int32) + base)
        if t < K:
            d = jnp.where(iota == amin, jnp.float32(jnp.inf), d)
    out_ref[0] = jnp.concatenate(cols, axis=1)


def _knn_indices(qv, vt):
    return pl.pallas_call(
        _knn_body,
        grid=(BS,),
        in_specs=[
            pl.BlockSpec((1, POOL, 3), lambda b: (b, 0, 0)),
            pl.BlockSpec((1, 3, V), lambda b: (b, 0, 0)),
        ],
        out_specs=pl.BlockSpec((1, POOL, K), lambda b: (b, 0, 0)),
        out_shape=jax.ShapeDtypeStruct((BS, POOL, K), jnp.int32),
    )(qv, vt)


def _sc_pool_body(fm_hbm, idx_hbm, out_hbm, idx_v, rows0, rows1, out_v,
                  sem0, sem1):
    wid = lax.axis_index("s") * NC + lax.axis_index("c")
    qbase = pl.multiple_of(wid * QPW, QPW)
    pltpu.sync_copy(idx_hbm.at[pl.ds(qbase * K, QPW * K)], idx_v)

    def _src(ci):
        off = pl.multiple_of(ci * ROWS_PER_CHUNK, ROWS_PER_CHUNK)
        return fm_hbm.at[idx_v.at[pl.ds(off, ROWS_PER_CHUNK)]]

    def _compute(ci, buf):
        def q_body(qq, inner_carry):
            row0 = qq * K
            for c in range(F // 16):
                sl = pl.ds(c * 16, 16)
                acc = buf[row0, sl]
                for r in range(1, K):
                    acc = jnp.maximum(acc, buf[row0 + r, sl])
                out_v[ci * CHUNK_Q + qq, sl] = acc
            return inner_carry

        lax.fori_loop(0, CHUNK_Q, q_body, 0)

    pltpu.make_async_copy(_src(0), rows0, sem0).start()

    def pair_body(i, carry):
        ci0 = i * 2
        ci1 = ci0 + 1
        pltpu.make_async_copy(_src(ci1), rows1, sem1).start()
        pltpu.make_async_copy(_src(ci0), rows0, sem0).wait()
        _compute(ci0, rows0)

        @pl.when(ci1 + 1 < N_CHUNK)
        def _():
            pltpu.make_async_copy(_src(ci1 + 1), rows0, sem0).start()

        pltpu.make_async_copy(_src(ci1), rows1, sem1).wait()
        _compute(ci1, rows1)
        return carry

    lax.fori_loop(0, N_CHUNK // 2, pair_body, 0)
    pltpu.sync_copy(out_v, out_hbm.at[pl.ds(qbase, QPW)])


@functools.cache
def _make_sc_pool():
    return functools.partial(
        pl.kernel,
        out_type=jax.ShapeDtypeStruct((NQ, F), jnp.float32),
        mesh=plsc.VectorSubcoreMesh(core_axis_name="c", subcore_axis_name="s"),
        scratch_types=[
            pltpu.VMEM((QPW * K,), jnp.int32),
            pltpu.VMEM((ROWS_PER_CHUNK, F), jnp.float32),
            pltpu.VMEM((ROWS_PER_CHUNK, F), jnp.float32),
            pltpu.VMEM((QPW, F), jnp.float32),
            pltpu.SemaphoreType.DMA,
            pltpu.SemaphoreType.DMA,
        ],
    )(_sc_pool_body)


def kernel(vertices, feature_map):
    sample_idx = jax.random.permutation(jax.random.key(42), V)[:POOL]
    qv = jnp.take(vertices, sample_idx, axis=1)
    vt = vertices.transpose(0, 2, 1)
    gidx = _knn_indices(qv, vt)
    fm_flat = feature_map.reshape(BS * V, F)
    idx_flat = gidx.reshape(NQ * K)
    pooled = _make_sc_pool()(fm_flat, idx_flat)
    return (qv, pooled.reshape(BS, POOL, F))
